# Initial kernel scaffold; baseline (speedup 1.0000x reference)
import jax
import jax.numpy as jnp
from jax.experimental import pallas as pl


def kernel(v_fea, t_emb, ef, W):
    raise NotImplementedError("write your pallas kernel here")


# speedup vs baseline: 2.0911x; 2.0911x over previous
import functools

import jax
import jax.numpy as jnp
from jax import lax
from jax.experimental import pallas as pl
from jax.experimental.pallas import tpu as pltpu
from jax.experimental.pallas import tpu_sc as plsc

L = 16


def _wq_body(v_ref, t_ref, w1_ref, w2_ref, o_ref):
    o_ref[...] = (
        jnp.dot(v_ref[...], w1_ref[...], preferred_element_type=jnp.float32)
        + jnp.dot(t_ref[...], w2_ref[...], preferred_element_type=jnp.float32)
    )


def _make_wq(NP, D, BN):
    return pl.pallas_call(
        _wq_body,
        grid=(NP // BN,),
        in_specs=[
            pl.BlockSpec((BN, D), lambda i: (i, 0)),
            pl.BlockSpec((BN, D), lambda i: (i, 0)),
            pl.BlockSpec((D, 2 * D), lambda i: (0, 0)),
            pl.BlockSpec((D, 2 * D), lambda i: (0, 0)),
        ],
        out_specs=pl.BlockSpec((BN, 2 * D), lambda i: (i, 0)),
        out_shape=jax.ShapeDtypeStruct((NP, 2 * D), jnp.float32),
    )


def _make_sc(K, D, NP, B):
    NW = 32
    rows_w = NP // NW
    nblocks = rows_w // B
    E = B * K
    C = D // L

    mesh = plsc.VectorSubcoreMesh(
        core_axis_name="c", subcore_axis_name="s", num_cores=2, num_subcores=16)

    @functools.partial(
        pl.kernel,
        out_type=jax.ShapeDtypeStruct((NP, D), jnp.float32),
        mesh=mesh,
        scratch_types=[
            pltpu.VMEM((E,), jnp.int32),
            pltpu.VMEM((E, D), jnp.float32),
            pltpu.VMEM((E, D), jnp.float32),
            pltpu.VMEM((B, 2 * D), jnp.float32),
            pltpu.VMEM((B, D), jnp.float32),
            pltpu.SemaphoreType.DMA,
            pltpu.SemaphoreType.DMA,
        ],
    )
    def sc_kernel(v_hbm, t_hbm, wq_hbm, ef_hbm, out_hbm,
                  idx_v, vrows, trows, wq_v, out_v, sem1, sem2):
        wid = lax.axis_index("s") * 2 + lax.axis_index("c")
        row0 = wid * rows_w
        iota = lax.iota(jnp.int32, L)

        def allreduce(x, op):
            for s in (8, 4, 2, 1):
                perm = jnp.bitwise_xor(iota, s)
                x = op(x, x.at[perm].get(mode="promise_in_bounds"))
            return x

        @pl.loop(0, nblocks)
        def _block(b):
            n0 = row0 + b * B
            pltpu.sync_copy(ef_hbm.at[pl.ds(n0 * K, E)], idx_v)
            pltpu.sync_copy(wq_hbm.at[pl.ds(n0, B)], wq_v)
            pltpu.async_copy(v_hbm.at[idx_v], vrows, sem1).wait()
            pltpu.async_copy(t_hbm.at[idx_v], trows, sem2).wait()
            for r in range(B):
                wv = [wq_v[r, pl.ds(c * L, L)] for c in range(2 * C)]
                e0 = r * K

                def score_body(k, carry, e0=e0, wv=wv):
                    s0, s1 = carry
                    row = e0 + k
                    acc = vrows[row, pl.ds(0, L)] * wv[0]
                    for c in range(1, C):
                        acc = acc + vrows[row, pl.ds(c * L, L)] * wv[c]
                    for c in range(C):
                        acc = acc + trows[row, pl.ds(c * L, L)] * wv[C + c]
                    sc = allreduce(acc, jnp.add)
                    s0 = jnp.where(iota == k, sc, s0)
                    s1 = jnp.where(iota == (k - L), sc, s1)
                    return s0, s1

                neg = jnp.full((L,), -3e38, jnp.float32)
                s0, s1 = lax.fori_loop(0, K, score_body, (neg, neg), unroll=4)
                m = allreduce(jnp.maximum(s0, s1), jnp.maximum)
                p0 = jnp.exp(s0 - m)
                p1 = jnp.exp(s1 - m)
                inv = 1.0 / allreduce(p0 + p1, jnp.add)
                w0 = p0 * inv
                w1 = p1 * inv

                def wsum_body(k, carry, e0=e0, w0=w0, w1=w1):
                    row = e0 + k
                    kl = jnp.full((L,), lax.bitwise_and(k, L - 1), jnp.int32)
                    wspl = jnp.where(
                        k < L,
                        w0.at[kl].get(mode="promise_in_bounds"),
                        w1.at[kl].get(mode="promise_in_bounds"))
                    return tuple(carry[c] + wspl * vrows[row, pl.ds(c * L, L)]
                                 for c in range(C))

                zero = jnp.zeros((L,), jnp.float32)
                accs = lax.fori_loop(0, K, wsum_body, (zero,) * C, unroll=4)
                for c in range(C):
                    out_v[r, pl.ds(c * L, L)] = accs[c]
            pltpu.sync_copy(out_v, out_hbm.at[pl.ds(n0, B)])

    return sc_kernel


def kernel(v_fea, t_emb, ef, W):
    N, D = v_fea.shape
    K = ef.shape[1]
    NW, B = 32, 4
    NP = ((N + NW * B - 1) // (NW * B)) * (NW * B)
    pad = NP - N
    vp = jnp.pad(v_fea, ((0, pad), (0, 0)))
    tp = jnp.pad(t_emb, ((0, pad), (0, 0)))
    W1 = W[:, :D].T
    W2 = W[:, D:].T
    BN = 1024 if NP % 1024 == 0 else 128
    wq = _make_wq(NP, D, BN)(vp, tp, W1, W2)
    efp = jnp.pad(ef.astype(jnp.int32), ((0, pad), (0, 0))).reshape(-1)
    out = _make_sc(K, D, NP, B)(v_fea, t_emb, wq, efp)
    return out[:N]


# speedup vs baseline: 2.7109x; 1.2964x over previous
import functools

import jax
import jax.numpy as jnp
from jax import lax
from jax.experimental import pallas as pl
from jax.experimental.pallas import tpu as pltpu
from jax.experimental.pallas import tpu_sc as plsc

L = 16
_HIMASK = -65536


def _wq_body(v_ref, t_ref, w1_ref, w2_ref, o_ref):
    o_ref[...] = (
        jnp.dot(v_ref[...], w1_ref[...], preferred_element_type=jnp.float32)
        + jnp.dot(t_ref[...], w2_ref[...], preferred_element_type=jnp.float32)
    )


def _make_wq(NP, D, BN):
    return pl.pallas_call(
        _wq_body,
        grid=(NP // BN,),
        in_specs=[
            pl.BlockSpec((BN, D), lambda i: (i, 0)),
            pl.BlockSpec((BN, D), lambda i: (i, 0)),
            pl.BlockSpec((D, 2 * D), lambda i: (0, 0)),
            pl.BlockSpec((D, 2 * D), lambda i: (0, 0)),
        ],
        out_specs=pl.BlockSpec((BN, 2 * D), lambda i: (i, 0)),
        out_shape=jax.ShapeDtypeStruct((NP, 2 * D), jnp.float32),
    )


def _bf16_packed_i32(x):
    R, Cc = x.shape
    xr = x.reshape(R, Cc // 32, 2, 16).transpose(0, 1, 3, 2)
    xb = xr.reshape(R, Cc).astype(jnp.bfloat16)
    return jax.lax.bitcast_convert_type(
        xb.reshape(R, Cc // 2, 2), jnp.int32)


def _make_sc(K, D, NP, B):
    NW = 32
    rows_w = NP // NW
    nblocks = rows_w // B
    E = B * K
    G = D // 32

    mesh = plsc.VectorSubcoreMesh(
        core_axis_name="c", subcore_axis_name="s", num_cores=2, num_subcores=16)

    @functools.partial(
        pl.kernel,
        out_type=jax.ShapeDtypeStruct((NP, D), jnp.float32),
        mesh=mesh,
        scratch_types=[
            pltpu.VMEM((E,), jnp.int32),
            pltpu.VMEM((E, D), jnp.int32),
            pltpu.VMEM((B, D), jnp.int32),
            pltpu.VMEM((B, D), jnp.float32),
            pltpu.SemaphoreType.DMA,
        ],
    )
    def sc_kernel(q_hbm, wq_hbm, ef_hbm, out_hbm,
                  idx_v, qrows, wq_v, out_v, sem1):
        wid = lax.axis_index("s") * 2 + lax.axis_index("c")
        row0 = wid * rows_w
        iota = lax.iota(jnp.int32, L)

        def allreduce(x, op):
            for s in (8, 4, 2, 1):
                perm = jnp.bitwise_xor(iota, s)
                x = op(x, x.at[perm].get(mode="promise_in_bounds"))
            return x

        def ld2(ref, row, g):
            u = ref[row, pl.ds(g * L, L)]
            lo = lax.bitcast_convert_type(lax.shift_left(u, 16), jnp.float32)
            hi = lax.bitcast_convert_type(
                lax.bitwise_and(u, _HIMASK), jnp.float32)
            return lo, hi

        @pl.loop(0, nblocks)
        def _block(b):
            n0 = row0 + b * B
            pltpu.sync_copy(ef_hbm.at[pl.ds(n0 * K, E)], idx_v)
            pltpu.sync_copy(wq_hbm.at[pl.ds(n0, B)], wq_v)
            pltpu.async_copy(q_hbm.at[idx_v], qrows, sem1).wait()
            for r in range(B):
                wvs = []
                for g in range(2 * G):
                    wvs += list(ld2(wq_v, r, g))
                e0 = r * K

                def score_body(k, carry, e0=e0, wvs=wvs):
                    s0, s1 = carry
                    row = e0 + k
                    lo, hi = ld2(qrows, row, 0)
                    acc = lo * wvs[0] + hi * wvs[1]
                    for g in range(1, 2 * G):
                        lo, hi = ld2(qrows, row, g)
                        acc = acc + lo * wvs[2 * g] + hi * wvs[2 * g + 1]
                    sc = allreduce(acc, jnp.add)
                    s0 = jnp.where(iota == k, sc, s0)
                    s1 = jnp.where(iota == (k - L), sc, s1)
                    return s0, s1

                neg = jnp.full((L,), -3e38, jnp.float32)
                s0, s1 = lax.fori_loop(0, K, score_body, (neg, neg), unroll=4)
                m = allreduce(jnp.maximum(s0, s1), jnp.maximum)
                p0 = jnp.exp(s0 - m)
                p1 = jnp.exp(s1 - m)
                inv = 1.0 / allreduce(p0 + p1, jnp.add)
                w0 = p0 * inv
                w1 = p1 * inv

                def wsum_body(k, carry, e0=e0, w0=w0, w1=w1):
                    row = e0 + k
                    kl = jnp.full((L,), lax.bitwise_and(k, L - 1), jnp.int32)
                    wspl = jnp.where(
                        k < L,
                        w0.at[kl].get(mode="promise_in_bounds"),
                        w1.at[kl].get(mode="promise_in_bounds"))
                    acc = []
                    for g in range(G):
                        lo, hi = ld2(qrows, row, g)
                        acc.append(carry[2 * g] + wspl * lo)
                        acc.append(carry[2 * g + 1] + wspl * hi)
                    return tuple(acc)

                zero = jnp.zeros((L,), jnp.float32)
                accs = lax.fori_loop(0, K, wsum_body, (zero,) * (2 * G),
                                     unroll=4)
                for j in range(2 * G):
                    out_v[r, pl.ds(j * L, L)] = accs[j]
            pltpu.sync_copy(out_v, out_hbm.at[pl.ds(n0, B)])

    return sc_kernel


def kernel(v_fea, t_emb, ef, W):
    N, D = v_fea.shape
    K = ef.shape[1]
    NW, B = 32, 4
    NP = ((N + NW * B - 1) // (NW * B)) * (NW * B)
    pad = NP - N
    vp = jnp.pad(v_fea, ((0, pad), (0, 0)))
    tp = jnp.pad(t_emb, ((0, pad), (0, 0)))
    W1 = W[:, :D].T
    W2 = W[:, D:].T
    BN = 1024 if NP % 1024 == 0 else 128
    wq = _make_wq(NP, D, BN)(vp, tp, W1, W2)
    qb = jnp.concatenate(
        [_bf16_packed_i32(v_fea), _bf16_packed_i32(t_emb)], axis=1)
    wqb = _bf16_packed_i32(wq)
    efp = jnp.pad(ef.astype(jnp.int32), ((0, pad), (0, 0))).reshape(-1)
    out = _make_sc(K, D, NP, B)(qb, wqb, efp)
    return out[:N]


# speedup vs baseline: 2.7827x; 1.0265x over previous
import functools

import jax
import jax.numpy as jnp
from jax import lax
from jax.experimental import pallas as pl
from jax.experimental.pallas import tpu as pltpu
from jax.experimental.pallas import tpu_sc as plsc

L = 16
_HIMASK = -65536


def _wq_body(v_ref, t_ref, w1_ref, w2_ref, o_ref):
    o_ref[...] = (
        jnp.dot(v_ref[...], w1_ref[...], preferred_element_type=jnp.float32)
        + jnp.dot(t_ref[...], w2_ref[...], preferred_element_type=jnp.float32)
    )


def _make_wq(NP, D, BN):
    return pl.pallas_call(
        _wq_body,
        grid=(NP // BN,),
        in_specs=[
            pl.BlockSpec((BN, D), lambda i: (i, 0)),
            pl.BlockSpec((BN, D), lambda i: (i, 0)),
            pl.BlockSpec((D, 2 * D), lambda i: (0, 0)),
            pl.BlockSpec((D, 2 * D), lambda i: (0, 0)),
        ],
        out_specs=pl.BlockSpec((BN, 2 * D), lambda i: (i, 0)),
        out_shape=jax.ShapeDtypeStruct((NP, 2 * D), jnp.float32),
    )


def _bf16_packed_i32(x):
    R, Cc = x.shape
    xr = x.reshape(R, Cc // 32, 2, 16).transpose(0, 1, 3, 2)
    xb = xr.reshape(R, Cc).astype(jnp.bfloat16)
    return jax.lax.bitcast_convert_type(
        xb.reshape(R, Cc // 2, 2), jnp.int32)


def _make_sc(K, D, NP, B):
    NW = 32
    rows_w = NP // NW
    nblocks = rows_w // B
    E = B * K
    IW = 128
    RPB = E // IW
    G = D // 32

    mesh = plsc.VectorSubcoreMesh(
        core_axis_name="c", subcore_axis_name="s", num_cores=2, num_subcores=16)

    @functools.partial(
        pl.kernel,
        out_type=jax.ShapeDtypeStruct((NP, D), jnp.float32),
        mesh=mesh,
        scratch_types=[
            pltpu.VMEM((nblocks * RPB, IW), jnp.int32),
            pltpu.VMEM((rows_w, D), jnp.int32),
            pltpu.VMEM((E, D), jnp.int32),
            pltpu.VMEM((E, D), jnp.int32),
            pltpu.VMEM((B, D), jnp.float32),
            pltpu.VMEM((B, D), jnp.float32),
            pltpu.SemaphoreType.DMA,
            pltpu.SemaphoreType.DMA,
        ],
    )
    def sc_kernel(q_hbm, wq_hbm, ef_hbm, out_hbm,
                  idx_all, wq_all, qrA, qrB, outA, outB, semA, semB):
        wid = lax.axis_index("s") * 2 + lax.axis_index("c")
        row0 = pl.multiple_of(wid * rows_w, 8)
        iota = lax.iota(jnp.int32, L)

        def allreduce(x, op):
            for s in (8, 4, 2, 1):
                perm = jnp.bitwise_xor(iota, s)
                x = op(x, x.at[perm].get(mode="promise_in_bounds"))
            return x

        def ld2(ref, row, g):
            u = ref[row, pl.ds(g * L, L)]
            lo = lax.bitcast_convert_type(lax.shift_left(u, 16), jnp.float32)
            hi = lax.bitcast_convert_type(
                lax.bitwise_and(u, _HIMASK), jnp.float32)
            return lo, hi

        def compute_block(b, qrows, out_v):
            for r in range(B):
                rg = b * B + r
                wvs = []
                for g in range(2 * G):
                    wvs += list(ld2(wq_all, rg, g))
                e0 = r * K

                def score_body(k, carry, e0=e0, wvs=wvs):
                    s0, s1 = carry
                    row = e0 + k
                    lo, hi = ld2(qrows, row, 0)
                    acc = lo * wvs[0] + hi * wvs[1]
                    for g in range(1, 2 * G):
                        lo, hi = ld2(qrows, row, g)
                        acc = acc + lo * wvs[2 * g] + hi * wvs[2 * g + 1]
                    sc = allreduce(acc, jnp.add)
                    s0 = jnp.where(iota == k, sc, s0)
                    s1 = jnp.where(iota == (k - L), sc, s1)
                    return s0, s1

                neg = jnp.full((L,), -3e38, jnp.float32)
                s0, s1 = lax.fori_loop(0, K, score_body, (neg, neg), unroll=4)
                m = allreduce(jnp.maximum(s0, s1), jnp.maximum)
                p0 = jnp.exp(s0 - m)
                p1 = jnp.exp(s1 - m)
                inv = 1.0 / allreduce(p0 + p1, jnp.add)
                w0 = p0 * inv
                w1 = p1 * inv

                def wsum_body(k, carry, e0=e0, w0=w0, w1=w1):
                    row = e0 + k
                    kl = jnp.full((L,), lax.bitwise_and(k, L - 1), jnp.int32)
                    wspl = jnp.where(
                        k < L,
                        w0.at[kl].get(mode="promise_in_bounds"),
                        w1.at[kl].get(mode="promise_in_bounds"))
                    acc = []
                    for g in range(G):
                        lo, hi = ld2(qrows, row, g)
                        acc.append(carry[2 * g] + wspl * lo)
                        acc.append(carry[2 * g + 1] + wspl * hi)
                    return tuple(acc)

                zero = jnp.zeros((L,), jnp.float32)
                accs = lax.fori_loop(0, K, wsum_body, (zero,) * (2 * G),
                                     unroll=4)
                for j in range(2 * G):
                    out_v[r, pl.ds(j * L, L)] = accs[j]
            o0 = pl.multiple_of(row0 + b * B, 8)
            pltpu.sync_copy(out_v, out_hbm.at[pl.ds(o0, B)])

        def issue(b, qr, sem):
            for h in range(RPB):
                pltpu.async_copy(q_hbm.at[idx_all.at[b * RPB + h]],
                                 qr.at[pl.ds(h * IW, IW)], sem)

        def drain(qr, sem):
            for h in range(RPB):
                pltpu.make_async_copy(q_hbm.at[pl.ds(0, IW)],
                                      qr.at[pl.ds(h * IW, IW)], sem).wait()

        i0 = pl.multiple_of(wid * (nblocks * RPB), 8)
        pltpu.sync_copy(ef_hbm.at[pl.ds(i0, nblocks * RPB)], idx_all)
        pltpu.sync_copy(wq_hbm.at[pl.ds(row0, rows_w)], wq_all)
        issue(0, qrA, semA)

        @pl.loop(0, nblocks // 2)
        def _pair(bb):
            b0 = bb * 2
            issue(b0 + 1, qrB, semB)
            drain(qrA, semA)
            compute_block(b0, qrA, outA)

            @pl.when(b0 + 2 < nblocks)
            def _():
                issue(b0 + 2, qrA, semA)

            drain(qrB, semB)
            compute_block(b0 + 1, qrB, outB)

    return sc_kernel


def kernel(v_fea, t_emb, ef, W):
    N, D = v_fea.shape
    K = ef.shape[1]
    NW, B = 32, 8
    NP = ((N + NW * B - 1) // (NW * B)) * (NW * B)
    pad = NP - N
    vp = jnp.pad(v_fea, ((0, pad), (0, 0)))
    tp = jnp.pad(t_emb, ((0, pad), (0, 0)))
    W1 = W[:, :D].T
    W2 = W[:, D:].T
    BN = 1024 if NP % 1024 == 0 else 128
    wq = _make_wq(NP, D, BN)(vp, tp, W1, W2)
    qb = jnp.concatenate(
        [_bf16_packed_i32(v_fea), _bf16_packed_i32(t_emb)], axis=1)
    wqb = _bf16_packed_i32(wq)
    efp = jnp.pad(ef.astype(jnp.int32), ((0, pad), (0, 0))).reshape(-1, 128)
    out = _make_sc(K, D, NP, B)(qb, wqb, efp)
    return out[:N]


# speedup vs baseline: 2.8522x; 1.0250x over previous
import functools

import jax
import jax.numpy as jnp
from jax import lax
from jax.experimental import pallas as pl
from jax.experimental.pallas import tpu as pltpu
from jax.experimental.pallas import tpu_sc as plsc

L = 16
_HIMASK = -65536


def _wq_body(v_ref, t_ref, w1_ref, w2_ref, o_ref):
    o_ref[...] = (
        jnp.dot(v_ref[...], w1_ref[...], preferred_element_type=jnp.float32)
        + jnp.dot(t_ref[...], w2_ref[...], preferred_element_type=jnp.float32)
    )


def _make_wq(NP, D, BN):
    return pl.pallas_call(
        _wq_body,
        grid=(NP // BN,),
        in_specs=[
            pl.BlockSpec((BN, D), lambda i: (i, 0)),
            pl.BlockSpec((BN, D), lambda i: (i, 0)),
            pl.BlockSpec((D, 2 * D), lambda i: (0, 0)),
            pl.BlockSpec((D, 2 * D), lambda i: (0, 0)),
        ],
        out_specs=pl.BlockSpec((BN, 2 * D), lambda i: (i, 0)),
        out_shape=jax.ShapeDtypeStruct((NP, 2 * D), jnp.float32),
    )


def _bf16_packed_i32(x):
    R, Cc = x.shape
    xr = x.reshape(R, Cc // 32, 2, 16).transpose(0, 1, 3, 2)
    xb = xr.reshape(R, Cc).astype(jnp.bfloat16)
    return jax.lax.bitcast_convert_type(
        xb.reshape(R, Cc // 2, 2), jnp.int32)


def _make_sc(K, D, NP, B):
    NW = 32
    rows_w = NP // NW
    nblocks = rows_w // B
    E = B * K
    IW = 128
    RPB = E // IW
    G = D // 32

    mesh = plsc.VectorSubcoreMesh(
        core_axis_name="c", subcore_axis_name="s", num_cores=2, num_subcores=16)

    @functools.partial(
        pl.kernel,
        out_type=jax.ShapeDtypeStruct((NP, D), jnp.float32),
        mesh=mesh,
        scratch_types=[
            pltpu.VMEM((nblocks * RPB, IW), jnp.int32),
            pltpu.VMEM((rows_w, D), jnp.int32),
            pltpu.VMEM((E, D), jnp.int32),
            pltpu.VMEM((E, D), jnp.int32),
            pltpu.VMEM((B, D), jnp.float32),
            pltpu.VMEM((B, D), jnp.float32),
            pltpu.SemaphoreType.DMA,
            pltpu.SemaphoreType.DMA,
        ],
    )
    def sc_kernel(q_hbm, wq_hbm, ef_hbm, out_hbm,
                  idx_all, wq_all, qrA, qrB, outA, outB, semA, semB):
        wid = lax.axis_index("s") * 2 + lax.axis_index("c")
        row0 = pl.multiple_of(wid * rows_w, 8)
        iota = lax.iota(jnp.int32, L)

        def allreduce(x, op):
            for s in (8, 4, 2, 1):
                perm = jnp.bitwise_xor(iota, s)
                x = op(x, x.at[perm].get(mode="promise_in_bounds"))
            return x

        def ld2(ref, row, g):
            u = ref[row, pl.ds(g * L, L)]
            lo = lax.bitcast_convert_type(lax.shift_left(u, 16), jnp.float32)
            hi = lax.bitcast_convert_type(
                lax.bitwise_and(u, _HIMASK), jnp.float32)
            return lo, hi

        def compute_block(b, qrows, out_v):
            for r in range(B):
                rg = b * B + r
                wvs = []
                for g in range(2 * G):
                    wvs += list(ld2(wq_all, rg, g))
                e0 = r * K

                neg = jnp.full((L,), -3e38, jnp.float32)

                @plsc.parallel_loop(0, K, unroll=4, carry=(neg, neg))
                def score_carry(k, carry, e0=e0, wvs=wvs):
                    s0, s1 = carry
                    row = e0 + k
                    a = [None, None, None, None]
                    for g in range(2 * G):
                        lo, hi = ld2(qrows, row, g)
                        p = lo * wvs[2 * g] + hi * wvs[2 * g + 1]
                        j = g % 4
                        a[j] = p if a[j] is None else a[j] + p
                    acc = (a[0] + a[1]) + (a[2] + a[3])
                    sc = allreduce(acc, jnp.add)
                    s0 = jnp.where(iota == k, sc, s0)
                    s1 = jnp.where(iota == (k - L), sc, s1)
                    return s0, s1

                s0, s1 = score_carry
                m = allreduce(jnp.maximum(s0, s1), jnp.maximum)
                p0 = jnp.exp(s0 - m)
                p1 = jnp.exp(s1 - m)
                inv = 1.0 / allreduce(p0 + p1, jnp.add)
                w0 = p0 * inv
                w1 = p1 * inv

                zero = jnp.zeros((L,), jnp.float32)

                @plsc.parallel_loop(0, K, unroll=4, carry=(zero,) * (2 * G))
                def wsum_carry(k, carry, e0=e0, w0=w0, w1=w1):
                    row = e0 + k
                    kl = jnp.full((L,), lax.bitwise_and(k, L - 1), jnp.int32)
                    wspl = jnp.where(
                        k < L,
                        w0.at[kl].get(mode="promise_in_bounds"),
                        w1.at[kl].get(mode="promise_in_bounds"))
                    acc = []
                    for g in range(G):
                        lo, hi = ld2(qrows, row, g)
                        acc.append(carry[2 * g] + wspl * lo)
                        acc.append(carry[2 * g + 1] + wspl * hi)
                    return tuple(acc)

                accs = wsum_carry
                for j in range(2 * G):
                    out_v[r, pl.ds(j * L, L)] = accs[j]
            o0 = pl.multiple_of(row0 + b * B, 8)
            pltpu.sync_copy(out_v, out_hbm.at[pl.ds(o0, B)])

        def issue(b, qr, sem):
            for h in range(RPB):
                pltpu.async_copy(q_hbm.at[idx_all.at[b * RPB + h]],
                                 qr.at[pl.ds(h * IW, IW)], sem)

        def drain(qr, sem):
            for h in range(RPB):
                pltpu.make_async_copy(q_hbm.at[pl.ds(0, IW)],
                                      qr.at[pl.ds(h * IW, IW)], sem).wait()

        i0 = pl.multiple_of(wid * (nblocks * RPB), 8)
        pltpu.sync_copy(ef_hbm.at[pl.ds(i0, nblocks * RPB)], idx_all)
        pltpu.sync_copy(wq_hbm.at[pl.ds(row0, rows_w)], wq_all)
        issue(0, qrA, semA)

        @pl.loop(0, nblocks // 2)
        def _pair(bb):
            b0 = bb * 2
            issue(b0 + 1, qrB, semB)
            drain(qrA, semA)
            compute_block(b0, qrA, outA)

            @pl.when(b0 + 2 < nblocks)
            def _():
                issue(b0 + 2, qrA, semA)

            drain(qrB, semB)
            compute_block(b0 + 1, qrB, outB)

    return sc_kernel


def kernel(v_fea, t_emb, ef, W):
    N, D = v_fea.shape
    K = ef.shape[1]
    NW, B = 32, 8
    NP = ((N + NW * B - 1) // (NW * B)) * (NW * B)
    pad = NP - N
    vp = jnp.pad(v_fea, ((0, pad), (0, 0)))
    tp = jnp.pad(t_emb, ((0, pad), (0, 0)))
    W1 = W[:, :D].T
    W2 = W[:, D:].T
    BN = 1024 if NP % 1024 == 0 else 128
    wq = _make_wq(NP, D, BN)(vp, tp, W1, W2)
    qb = jnp.concatenate(
        [_bf16_packed_i32(v_fea), _bf16_packed_i32(t_emb)], axis=1)
    wqb = _bf16_packed_i32(wq)
    efp = jnp.pad(ef.astype(jnp.int32), ((0, pad), (0, 0))).reshape(-1, 128)
    out = _make_sc(K, D, NP, B)(qb, wqb, efp)
    return out[:N]


# speedup vs baseline: 5.4223x; 1.9011x over previous
import functools

import jax
import jax.numpy as jnp
from jax import lax
from jax.experimental import pallas as pl
from jax.experimental.pallas import tpu as pltpu
from jax.experimental.pallas import tpu_sc as plsc

L = 16
_HIMASK = -65536


def _wq_body(v_ref, t_ref, w1_ref, w2_ref, o_ref):
    o_ref[...] = (
        jnp.dot(v_ref[...], w1_ref[...], preferred_element_type=jnp.float32)
        + jnp.dot(t_ref[...], w2_ref[...], preferred_element_type=jnp.float32)
    )


def _make_wq(NP, D, BN):
    return pl.pallas_call(
        _wq_body,
        grid=(NP // BN,),
        in_specs=[
            pl.BlockSpec((BN, D), lambda i: (i, 0)),
            pl.BlockSpec((BN, D), lambda i: (i, 0)),
            pl.BlockSpec((D, 2 * D), lambda i: (0, 0)),
            pl.BlockSpec((D, 2 * D), lambda i: (0, 0)),
        ],
        out_specs=pl.BlockSpec((BN, 2 * D), lambda i: (i, 0)),
        out_shape=jax.ShapeDtypeStruct((NP, 2 * D), jnp.float32),
    )


def _bf16_packed_i32(x):
    R, Cc = x.shape
    xr = x.reshape(R, Cc // 32, 2, 16).transpose(0, 1, 3, 2)
    xb = xr.reshape(R, Cc).astype(jnp.bfloat16)
    return jax.lax.bitcast_convert_type(
        xb.reshape(R, Cc // 2, 2), jnp.int32)


def _make_sc(K, D, NP, B):
    NW = 32
    rows_w = NP // NW
    nblocks = rows_w // B
    E = B * K
    IW = 128
    RPB = E // IW
    G = D // 32

    mesh = plsc.VectorSubcoreMesh(
        core_axis_name="c", subcore_axis_name="s", num_cores=2, num_subcores=16)

    @functools.partial(
        pl.kernel,
        out_type=jax.ShapeDtypeStruct((NP, D), jnp.float32),
        mesh=mesh,
        scratch_types=[
            pltpu.VMEM((nblocks * RPB, IW), jnp.int32),
            pltpu.VMEM((2 * B, D), jnp.int32),
            pltpu.VMEM((E, D), jnp.int32),
            pltpu.VMEM((E, D), jnp.int32),
            pltpu.VMEM((2 * B, D), jnp.float32),
            pltpu.VMEM_SHARED((NP, D), jnp.int32),
            pltpu.SemaphoreType.DMA,
            pltpu.SemaphoreType.DMA,
        ],
    )
    def sc_kernel(q_hbm, wq_hbm, ef_hbm, out_hbm,
                  idx_all, wq_p, qrA, qrB, out_p, qsh, semA, semB):
        wid = lax.axis_index("s") * 2 + lax.axis_index("c")
        row0 = pl.multiple_of(wid * rows_w, 8)
        iota = lax.iota(jnp.int32, L)

        def allreduce(x, op):
            for s in (8, 4, 2, 1):
                perm = jnp.bitwise_xor(iota, s)
                x = op(x, x.at[perm].get(mode="promise_in_bounds"))
            return x

        def ld2(ref, row, g):
            u = ref[row, pl.ds(g * L, L)]
            lo = lax.bitcast_convert_type(lax.shift_left(u, 16), jnp.float32)
            hi = lax.bitcast_convert_type(
                lax.bitwise_and(u, _HIMASK), jnp.float32)
            return lo, hi

        def compute_block(half, qrows):
            for r in range(B):
                rl = half * B + r
                wvs = []
                for g in range(2 * G):
                    wvs += list(ld2(wq_p, rl, g))
                e0 = r * K

                neg = jnp.full((L,), -3e38, jnp.float32)

                @plsc.parallel_loop(0, K, unroll=4, carry=(neg, neg))
                def score_carry(k, carry, e0=e0, wvs=wvs):
                    s0, s1 = carry
                    row = e0 + k
                    a = [None, None, None, None]
                    for g in range(2 * G):
                        lo, hi = ld2(qrows, row, g)
                        p = lo * wvs[2 * g] + hi * wvs[2 * g + 1]
                        j = g % 4
                        a[j] = p if a[j] is None else a[j] + p
                    acc = (a[0] + a[1]) + (a[2] + a[3])
                    sc = allreduce(acc, jnp.add)
                    s0 = jnp.where(iota == k, sc, s0)
                    s1 = jnp.where(iota == (k - L), sc, s1)
                    return s0, s1

                s0, s1 = score_carry
                m = allreduce(jnp.maximum(s0, s1), jnp.maximum)
                p0 = jnp.exp(s0 - m)
                p1 = jnp.exp(s1 - m)
                inv = 1.0 / allreduce(p0 + p1, jnp.add)
                w0 = p0 * inv
                w1 = p1 * inv

                zero = jnp.zeros((L,), jnp.float32)

                @plsc.parallel_loop(0, K, unroll=4, carry=(zero,) * (2 * G))
                def wsum_carry(k, carry, e0=e0, w0=w0, w1=w1):
                    row = e0 + k
                    kl = jnp.full((L,), lax.bitwise_and(k, L - 1), jnp.int32)
                    wspl = jnp.where(
                        k < L,
                        w0.at[kl].get(mode="promise_in_bounds"),
                        w1.at[kl].get(mode="promise_in_bounds"))
                    acc = []
                    for g in range(G):
                        lo, hi = ld2(qrows, row, g)
                        acc.append(carry[2 * g] + wspl * lo)
                        acc.append(carry[2 * g + 1] + wspl * hi)
                    return tuple(acc)

                accs = wsum_carry
                for j in range(2 * G):
                    out_p[rl, pl.ds(j * L, L)] = accs[j]

        def issue(b, qr, sem):
            for h in range(RPB):
                pltpu.async_copy(qsh.at[idx_all.at[b * RPB + h]],
                                 qr.at[pl.ds(h * IW, IW)], sem)

        def drain(qr, sem):
            for h in range(RPB):
                pltpu.make_async_copy(q_hbm.at[pl.ds(0, IW)],
                                      qr.at[pl.ds(h * IW, IW)], sem).wait()

        sub = lax.axis_index("s")
        shard = NP // 16
        t0 = pl.multiple_of(sub * shard, 8)
        pltpu.sync_copy(q_hbm.at[pl.ds(t0, shard)], qsh.at[pl.ds(t0, shard)])
        i0 = pl.multiple_of(wid * (nblocks * RPB), 8)
        pltpu.sync_copy(ef_hbm.at[pl.ds(i0, nblocks * RPB)], idx_all)
        plsc.subcore_barrier()
        issue(0, qrA, semA)
        issue(1, qrB, semB)

        @pl.loop(0, nblocks // 2)
        def _pair(p):
            b0 = p * 2
            w0r = pl.multiple_of(row0 + b0 * B, 8)
            pltpu.sync_copy(wq_hbm.at[pl.ds(w0r, 2 * B)], wq_p)
            drain(qrA, semA)
            compute_block(0, qrA)

            @pl.when(b0 + 2 < nblocks)
            def _():
                issue(b0 + 2, qrA, semA)

            drain(qrB, semB)
            compute_block(1, qrB)

            @pl.when(b0 + 3 < nblocks)
            def _():
                issue(b0 + 3, qrB, semB)

            pltpu.sync_copy(out_p, out_hbm.at[pl.ds(w0r, 2 * B)])

    return sc_kernel


def kernel(v_fea, t_emb, ef, W):
    N, D = v_fea.shape
    K = ef.shape[1]
    NW, B = 32, 4
    NP = ((N + 2 * NW * B - 1) // (2 * NW * B)) * (2 * NW * B)
    pad = NP - N
    vp = jnp.pad(v_fea, ((0, pad), (0, 0)))
    tp = jnp.pad(t_emb, ((0, pad), (0, 0)))
    W1 = W[:, :D].T
    W2 = W[:, D:].T
    BN = 1024 if NP % 1024 == 0 else 128
    wq = _make_wq(NP, D, BN)(vp, tp, W1, W2)
    qb = jnp.concatenate(
        [_bf16_packed_i32(v_fea), _bf16_packed_i32(t_emb)], axis=1)
    qb = jnp.pad(qb, ((0, pad), (0, 0)))
    wqb = _bf16_packed_i32(wq)
    efp = jnp.pad(ef.astype(jnp.int32), ((0, pad), (0, 0))).reshape(-1, 128)
    out = _make_sc(K, D, NP, B)(qb, wqb, efp)
    return out[:N]
